# Initial kernel scaffold; baseline (speedup 1.0000x reference)
import jax
import jax.numpy as jnp
from jax.experimental import pallas as pl


def kernel(token_embeddings, pos_table):
    raise NotImplementedError("write your pallas kernel here")


# speedup vs baseline: 3.2859x; 3.2859x over previous
import jax
import jax.numpy as jnp
from jax.experimental import pallas as pl

SEQ_BLOCK = 512


def _add_kernel(tok_ref, pos_ref, out_ref):
    out_ref[...] = tok_ref[...] + pos_ref[...][None, :, :]


def kernel(token_embeddings, pos_table):
    batch, seq_len, dims = token_embeddings.shape
    grid = (seq_len // SEQ_BLOCK,)
    return pl.pallas_call(
        _add_kernel,
        grid=grid,
        in_specs=[
            pl.BlockSpec((batch, SEQ_BLOCK, dims), lambda i: (0, i, 0)),
            pl.BlockSpec((SEQ_BLOCK, dims), lambda i: (i, 0)),
        ],
        out_specs=pl.BlockSpec((batch, SEQ_BLOCK, dims), lambda i: (0, i, 0)),
        out_shape=jax.ShapeDtypeStruct((batch, seq_len, dims), token_embeddings.dtype),
    )(token_embeddings, pos_table)
